# baseline (speedup 1.0000x reference)
import dataclasses
import functools

import jax
import jax.numpy as jnp
from jax import lax
from jax.experimental import pallas as pl
from jax.experimental.pallas import tpu as pltpu
from jax.experimental.pallas import tpu_sc as plsc

N = 10000
E = 320000
D = 128
D_EDGE = 16
HIDDEN = 64
AVG_NEIGH = 32.0

NW = 32
CH = 128
CG = 8
NG = 10
CJ = CG * NG
E_PAD = NW * CJ * CH
CHP = CH // 4
CH2 = 64
CJ2 = 160
NP = 80
CPB2 = 40
PACK = 4
BLK_PK = 640
W_WIDE = BLK_PK // 2
N_BLKS = E_PAD // (PACK * BLK_PK)
REAL_BLKS = E // (PACK * BLK_PK)
NSUB = 16
N_PAD = 10240
NPS = N_PAD // NSUB


def _matmul_body(a_ref, w_ref, o_ref):
    o_ref[...] = jnp.dot(a_ref[...], w_ref[...],
                         preferred_element_type=jnp.float32)


def _final_body(p_ref, w_ref, o_ref):
    o_ref[...] = jnp.dot(p_ref[0] + p_ref[1], w_ref[...],
                         preferred_element_type=jnp.float32)


def _mlp_body(ef_ref, w1_ref, w2_ref, w3_ref, w4_ref, o_ref):
    i = pl.program_id(0)
    ef0 = ef_ref[...]
    ef = jnp.concatenate(
        [ef0[k * BLK_PK:(k + 1) * BLK_PK, :] for k in range(PACK)], axis=1)
    h = jax.nn.silu(jnp.dot(ef, w1_ref[...],
                            preferred_element_type=jnp.float32))
    h = jax.nn.silu(jnp.dot(h, w2_ref[...],
                            preferred_element_type=jnp.float32))
    h = jax.nn.silu(jnp.dot(h, w3_ref[...],
                            preferred_element_type=jnp.float32))
    w = jnp.dot(h, w4_ref[...], preferred_element_type=jnp.float32)
    o_ref[...] = w * jnp.where(i < REAL_BLKS, 1.0, 0.0)


def _sc_body(x_hbm, wm_hbm, att_hbm, send_hbm, recv_hbm, zeros_hbm, out_hbm,
             idx_v, rows_v, wm_v, att_v, acc, sem_i, sg0, sg1, sw0, sw1,
             sa0, sa1):
    cid = lax.axis_index("c")
    sid = lax.axis_index("s")
    tile = cid * NSUB + sid
    nslice = pl.ds(sid * NPS, NPS)
    pltpu.sync_copy(zeros_hbm.at[nslice], acc.at[nslice])
    plsc.subcore_barrier()
    sgs = (sg0, sg1)
    sws = (sw0, sw1)
    sas = (sa0, sa1)

    def issue(j, s):
        b = j // CPB2
        rem = j - b * CPB2
        kseg = rem // 10
        c10 = rem - kseg * 10
        cw = pltpu.async_copy(
            wm_hbm.at[pl.ds(b * BLK_PK + c10 * CH2, CH2),
                      pl.ds(kseg * D, D)],
            wm_v.at[s], sws[s])
        p = (j // 2) % 2
        cg = pltpu.async_copy(x_hbm.at[idx_v.at[p, 0, j % 2]],
                              rows_v.at[s], sgs[s])
        ca = pltpu.async_copy(att_hbm.at[pl.ds(j * CH2, CH2)],
                              att_v.at[s], sas[s])
        return cw, cg, ca

    def process(j, s):
        pltpu.make_async_copy(wm_hbm.at[pl.ds(0, CH2), pl.ds(0, D)],
                              wm_v.at[s], sws[s]).wait()
        pltpu.make_async_copy(x_hbm.at[idx_v.at[0, 0, 0]],
                              rows_v.at[s], sgs[s]).wait()
        pltpu.make_async_copy(att_hbm.at[pl.ds(0, CH2)],
                              att_v.at[s], sas[s]).wait()

        @pl.loop(0, CH2)
        def _row(r):
            r16 = jnp.full((16,), r, dtype=jnp.int32)
            a16 = plsc.load_gather(att_v.at[s], [r16])
            for c in range(8):
                sl = pl.ds(c * 16, 16)
                rows_v[s, r, sl] = rows_v[s, r, sl] * wm_v[s, r, sl] * a16

        p = (j // 2) % 2
        pltpu.sync_copy(rows_v.at[s], acc.at[idx_v.at[p, 1, j % 2]],
                        add=True)

    pltpu.sync_copy(send_hbm.at[tile, 0], idx_v.at[0, 0])
    pltpu.sync_copy(recv_hbm.at[tile, 0], idx_v.at[0, 1])
    issue(tile * CJ2 + 0, 0)
    issue(tile * CJ2 + 1, 1)

    @pl.loop(0, NP)
    def _pair(jj):
        p = jj % 2
        j0 = tile * CJ2 + jj * 2

        @pl.when(jj < NP - 1)
        def _pf():
            ci1 = pltpu.async_copy(send_hbm.at[tile, jj + 1],
                                   idx_v.at[1 - p, 0], sem_i)
            ci2 = pltpu.async_copy(recv_hbm.at[tile, jj + 1],
                                   idx_v.at[1 - p, 1], sem_i)

        process(j0, 0)

        @pl.when(jj < NP - 1)
        def _n0():
            pltpu.make_async_copy(send_hbm.at[tile, 0], idx_v.at[0, 0],
                                  sem_i).wait()
            pltpu.make_async_copy(recv_hbm.at[tile, 0], idx_v.at[0, 1],
                                  sem_i).wait()
            issue(j0 + 2, 0)

        process(j0 + 1, 1)

        @pl.when(jj < NP - 1)
        def _n1():
            issue(j0 + 3, 1)

    plsc.subcore_barrier()
    pltpu.sync_copy(acc.at[nslice], out_hbm.at[cid, nslice])


def kernel(node_feats, edge_index, edge_attrs, edge_feats,
           W_up, W1, W2, W3, W4, W_lin):
    f32 = jnp.float32
    W_up_s = W_up * (1.0 / jnp.sqrt(f32(D)))
    W_lin_s = W_lin * (1.0 / (jnp.sqrt(f32(D)) * AVG_NEIGH))
    W1_s = W1 * (1.0 / jnp.sqrt(f32(D_EDGE)))
    W2_s = W2 * (1.0 / jnp.sqrt(f32(HIDDEN)))
    W3_s = W3 * (1.0 / jnp.sqrt(f32(HIDDEN)))
    W4_s = W4 * (1.0 / jnp.sqrt(f32(HIDDEN)))

    def bdiag(w, reps):
        rows, cols = w.shape
        out = jnp.zeros((rows * reps, cols * reps), f32)
        for k in range(reps):
            out = out.at[k * rows:(k + 1) * rows,
                         k * cols:(k + 1) * cols].set(w)
        return out

    W1b = bdiag(W1_s, PACK)
    W2b = bdiag(W2_s, PACK)
    W3b = bdiag(W3_s, PACK)
    W4b = bdiag(W4_s, PACK)

    x = pl.pallas_call(
        _matmul_body,
        out_shape=jax.ShapeDtypeStruct((N, D), f32),
    )(node_feats, W_up_s)

    full = lambda a: pl.BlockSpec(a.shape, lambda i: (0, 0))
    wm_pk = pl.pallas_call(
        _mlp_body,
        grid=(N_BLKS,),
        in_specs=[
            pl.BlockSpec((PACK * BLK_PK, D_EDGE),
                         lambda i: (jnp.minimum(i, REAL_BLKS - 1), 0)),
            full(W1b), full(W2b), full(W3b), full(W4b),
        ],
        out_specs=pl.BlockSpec((BLK_PK, D * PACK), lambda i: (i, 0)),
        out_shape=jax.ShapeDtypeStruct((E_PAD // PACK, D * PACK), f32),
    )(edge_feats, W1b, W2b, W3b, W4b)

    pad = E_PAD - E
    spread = jnp.arange(pad, dtype=jnp.int32) % N

    send = jnp.concatenate([edge_index[0], spread]).reshape(NW, NP, 2, CH2)
    recv = jnp.concatenate([edge_index[1], spread]).reshape(NW, NP, 2, CH2)
    att = jnp.concatenate([edge_attrs[:, 0], jnp.zeros((pad,), f32)])
    zeros = jnp.zeros((N_PAD, D), f32)

    mesh = plsc.VectorSubcoreMesh(core_axis_name="c", subcore_axis_name="s")
    cp = pltpu.CompilerParams()
    if "needs_layout_passes" in pltpu.CompilerParams.__dataclass_fields__:
        cp = dataclasses.replace(cp, needs_layout_passes=False)
    sc_call = functools.partial(
        pl.kernel,
        mesh=mesh,
        compiler_params=cp,
        out_type=jax.ShapeDtypeStruct((2, N_PAD, D), f32),
        scratch_types=[
            pltpu.VMEM((2, 2, 2, CH2), jnp.int32),
            pltpu.VMEM((2, CH2, D), f32),
            pltpu.VMEM((2, CH2, D), f32),
            pltpu.VMEM((2, CH2), f32),
            pltpu.VMEM_SHARED((N_PAD, D), f32),
            pltpu.SemaphoreType.DMA,
            pltpu.SemaphoreType.DMA,
            pltpu.SemaphoreType.DMA,
            pltpu.SemaphoreType.DMA,
            pltpu.SemaphoreType.DMA,
            pltpu.SemaphoreType.DMA,
            pltpu.SemaphoreType.DMA,
        ],
    )(_sc_body)
    partials = sc_call(x, wm_pk, att, send, recv, zeros)

    out_pad = pl.pallas_call(
        _final_body,
        out_shape=jax.ShapeDtypeStruct((N_PAD, D), f32),
    )(partials, W_lin_s)
    return out_pad[:N]


# speedup vs baseline: 1.4980x; 1.4980x over previous
import functools

import jax
import jax.numpy as jnp
from jax import lax
from jax.experimental import pallas as pl
from jax.experimental.pallas import tpu as pltpu
from jax.experimental.pallas import tpu_sc as plsc

N = 10000
E = 320000
D = 128
D_EDGE = 16
HIDDEN = 64
AVG_NEIGH = 32.0

NW = 32
CH = 128
CG = 8
NG = 10
CJ = CG * NG
E_PAD = NW * CJ * CH
CHP = CH // 4
CH2 = 64
CJ2 = 160
NP = 40
NSLAB = 2
E_SLAB = E_PAD // 2
SLAB_BLKS = 64
CJ2S = 80
CPB2 = 40
PACK = 4
BLK_PK = 640
W_WIDE = BLK_PK // 2
N_BLKS = E_PAD // (PACK * BLK_PK)
REAL_BLKS = E // (PACK * BLK_PK)
NSUB = 16
N_PAD = 10240
NPS = N_PAD // NSUB


def _matmul_body(a_ref, w_ref, o_ref):
    o_ref[...] = jnp.dot(a_ref[...], w_ref[...],
                         preferred_element_type=jnp.float32)


def _final_body(pa_ref, pb_ref, w_ref, o_ref):
    o_ref[...] = jnp.dot(pa_ref[0] + pa_ref[1] + pb_ref[0] + pb_ref[1],
                         w_ref[...], preferred_element_type=jnp.float32)


def _make_mlp_body(real_blks):
    def _mlp_body(ef_ref, a_ref, w1_ref, w2_ref, w3_ref, w4_ref, e4_ref,
                  o_ref):
        i = pl.program_id(0)
        ef0 = ef_ref[...]
        ef = jnp.concatenate(
            [ef0[k * BLK_PK:(k + 1) * BLK_PK, :] for k in range(PACK)],
            axis=1)
        h = jax.nn.silu(jnp.dot(ef, w1_ref[...],
                                preferred_element_type=jnp.float32))
        h = jax.nn.silu(jnp.dot(h, w2_ref[...],
                                preferred_element_type=jnp.float32))
        h = jax.nn.silu(jnp.dot(h, w3_ref[...],
                                preferred_element_type=jnp.float32))
        a0 = a_ref[...]
        a4 = jnp.concatenate(
            [a0[k * BLK_PK:(k + 1) * BLK_PK, :] for k in range(PACK)],
            axis=1)
        ab = jnp.dot(a4, e4_ref[...], preferred_element_type=jnp.float32)
        w = jnp.dot(h * ab, w4_ref[...], preferred_element_type=jnp.float32)
        o_ref[...] = w * jnp.where(i < real_blks, 1.0, 0.0)
    return _mlp_body


def _sc_body(x_hbm, wm_hbm, send_hbm, recv_hbm, zeros_hbm, out_hbm,
             idx_v, rows_v, wm_v, acc, sem_i, sg0, sg1, sw0, sw1):
    cid = lax.axis_index("c")
    sid = lax.axis_index("s")
    tile = cid * NSUB + sid
    nslice = pl.ds(sid * NPS, NPS)
    pltpu.sync_copy(zeros_hbm.at[nslice], acc.at[nslice])
    plsc.subcore_barrier()
    sgs = (sg0, sg1)
    sws = (sw0, sw1)

    def issue(j, s):
        b = j // CPB2
        rem = j - b * CPB2
        kseg = rem // 10
        c10 = rem - kseg * 10
        cw = pltpu.async_copy(
            wm_hbm.at[pl.ds(b * BLK_PK + c10 * CH2, CH2),
                      pl.ds(kseg * D, D)],
            wm_v.at[s], sws[s])
        p = (j // 2) % 2
        cg = pltpu.async_copy(x_hbm.at[idx_v.at[p, 0, j % 2]],
                              rows_v.at[s], sgs[s])
        return cw, cg

    def process(j, s):
        pltpu.make_async_copy(wm_hbm.at[pl.ds(0, CH2), pl.ds(0, D)],
                              wm_v.at[s], sws[s]).wait()
        pltpu.make_async_copy(x_hbm.at[idx_v.at[0, 0, 0]],
                              rows_v.at[s], sgs[s]).wait()

        @pl.loop(0, CH2)
        def _row(r):
            for c in range(8):
                sl = pl.ds(c * 16, 16)
                rows_v[s, r, sl] = rows_v[s, r, sl] * wm_v[s, r, sl]

        p = (j // 2) % 2
        pltpu.sync_copy(rows_v.at[s], acc.at[idx_v.at[p, 1, j % 2]],
                        add=True)

    pltpu.sync_copy(send_hbm.at[tile, 0], idx_v.at[0, 0])
    pltpu.sync_copy(recv_hbm.at[tile, 0], idx_v.at[0, 1])
    issue(tile * CJ2S + 0, 0)
    issue(tile * CJ2S + 1, 1)

    @pl.loop(0, NP)
    def _pair(jj):
        p = jj % 2
        j0 = tile * CJ2S + jj * 2

        @pl.when(jj < NP - 1)
        def _pf():
            ci1 = pltpu.async_copy(send_hbm.at[tile, jj + 1],
                                   idx_v.at[1 - p, 0], sem_i)
            ci2 = pltpu.async_copy(recv_hbm.at[tile, jj + 1],
                                   idx_v.at[1 - p, 1], sem_i)

        process(j0, 0)

        @pl.when(jj < NP - 1)
        def _n0():
            pltpu.make_async_copy(send_hbm.at[tile, 0], idx_v.at[0, 0],
                                  sem_i).wait()
            pltpu.make_async_copy(recv_hbm.at[tile, 0], idx_v.at[0, 1],
                                  sem_i).wait()
            issue(j0 + 2, 0)

        process(j0 + 1, 1)

        @pl.when(jj < NP - 1)
        def _n1():
            issue(j0 + 3, 1)

    plsc.subcore_barrier()
    pltpu.sync_copy(acc.at[nslice], out_hbm.at[cid, nslice])


def kernel(node_feats, edge_index, edge_attrs, edge_feats,
           W_up, W1, W2, W3, W4, W_lin):
    f32 = jnp.float32
    W_up_s = W_up * (1.0 / jnp.sqrt(f32(D)))
    W_lin_s = W_lin * (1.0 / (jnp.sqrt(f32(D)) * AVG_NEIGH))
    W1_s = W1 * (1.0 / jnp.sqrt(f32(D_EDGE)))
    W2_s = W2 * (1.0 / jnp.sqrt(f32(HIDDEN)))
    W3_s = W3 * (1.0 / jnp.sqrt(f32(HIDDEN)))
    W4_s = W4 * (1.0 / jnp.sqrt(f32(HIDDEN)))

    def bdiag(w, reps):
        rows, cols = w.shape
        out = jnp.zeros((rows * reps, cols * reps), f32)
        for k in range(reps):
            out = out.at[k * rows:(k + 1) * rows,
                         k * cols:(k + 1) * cols].set(w)
        return out

    W1b = bdiag(W1_s, PACK)
    W2b = bdiag(W2_s, PACK)
    W3b = bdiag(W3_s, PACK)
    W4b = bdiag(W4_s, PACK)
    E4 = bdiag(jnp.ones((1, HIDDEN), f32), PACK)

    x = pl.pallas_call(
        _matmul_body,
        out_shape=jax.ShapeDtypeStruct((N, D), f32),
    )(node_feats, W_up_s)

    full = lambda a: pl.BlockSpec(a.shape, lambda i: (0, 0))

    def mlp_slab(off_blks, real_blks):
        imap = lambda i: (jnp.minimum(i + off_blks, REAL_BLKS - 1), 0)
        return pl.pallas_call(
            _make_mlp_body(real_blks),
            grid=(SLAB_BLKS,),
            in_specs=[
                pl.BlockSpec((PACK * BLK_PK, D_EDGE), imap),
                pl.BlockSpec((PACK * BLK_PK, 1), imap),
                full(W1b), full(W2b), full(W3b), full(W4b), full(E4),
            ],
            out_specs=pl.BlockSpec((BLK_PK, D * PACK), lambda i: (i, 0)),
            out_shape=jax.ShapeDtypeStruct((E_SLAB // PACK, D * PACK), f32),
        )(edge_feats, edge_attrs, W1b, W2b, W3b, W4b, E4)

    wm_a = mlp_slab(0, SLAB_BLKS)
    wm_b = mlp_slab(SLAB_BLKS, 61)

    pad = E_PAD - E
    spread = jnp.arange(pad, dtype=jnp.int32) % N

    send = jnp.concatenate([edge_index[0], spread]).reshape(
        NSLAB, NW, NP, 2, CH2)
    recv = jnp.concatenate([edge_index[1], spread]).reshape(
        NSLAB, NW, NP, 2, CH2)
    zeros = jnp.zeros((N_PAD, D), f32)

    mesh = plsc.VectorSubcoreMesh(core_axis_name="c", subcore_axis_name="s")
    sc_call = functools.partial(
        pl.kernel,
        mesh=mesh,
        out_type=jax.ShapeDtypeStruct((2, N_PAD, D), f32),
        scratch_types=[
            pltpu.VMEM((2, 2, 2, CH2), jnp.int32),
            pltpu.VMEM((2, CH2, D), f32),
            pltpu.VMEM((2, CH2, D), f32),
            pltpu.VMEM_SHARED((N_PAD, D), f32),
            pltpu.SemaphoreType.DMA,
            pltpu.SemaphoreType.DMA,
            pltpu.SemaphoreType.DMA,
            pltpu.SemaphoreType.DMA,
            pltpu.SemaphoreType.DMA,
        ],
    )(_sc_body)
    pa = sc_call(x, wm_a, send[0], recv[0], zeros)
    pb = sc_call(x, wm_b, send[1], recv[1], zeros)

    out_pad = pl.pallas_call(
        _final_body,
        out_shape=jax.ShapeDtypeStruct((N_PAD, D), f32),
    )(pa, pb, W_lin_s)
    return out_pad[:N]
